# Initial kernel scaffold; baseline (speedup 1.0000x reference)
import jax
import jax.numpy as jnp
from jax.experimental import pallas as pl


def kernel(x, edge_index, edge_values, weight):
    raise NotImplementedError("write your pallas kernel here")


# speedup vs baseline: 3.9614x; 3.9614x over previous
import functools

import jax
import jax.numpy as jnp
from jax import lax
from jax.experimental import pallas as pl
from jax.experimental.pallas import tpu as pltpu
from jax.experimental.pallas import tpu_sc as plsc

N_NODES = 10000
N_PAD = 10240
D = 128
K = 128
NUM_CORES = 2
NUM_SUBCORES = 16
NW = NUM_CORES * NUM_SUBCORES
ROWS_PER_TILE = N_PAD // NUM_SUBCORES


def _matmul_body(x_ref, w_ref, o_ref):
    o_ref[...] = jnp.dot(x_ref[...], w_ref[...],
                         preferred_element_type=jnp.float32)


def _combine_body(p_ref, o_ref):
    o_ref[...] = p_ref[0] + p_ref[1]


@functools.lru_cache(maxsize=None)
def _make_spmm(n_chunks, per_tile):
    mesh = plsc.VectorSubcoreMesh(core_axis_name="c", subcore_axis_name="s")

    @functools.partial(
        pl.kernel,
        out_type=jax.ShapeDtypeStruct((NUM_CORES, N_PAD, D), jnp.float32),
        mesh=mesh,
        scratch_types=[
            pltpu.VMEM((K,), jnp.int32),
            pltpu.VMEM((K,), jnp.int32),
            pltpu.VMEM((K,), jnp.float32),
            pltpu.VMEM((K, D), jnp.float32),
            pltpu.VMEM_SHARED((N_PAD, D), jnp.float32),
            pltpu.SemaphoreType.DMA,
        ],
    )
    def spmm(src_hbm, dst_hbm, ev_hbm, xw_hbm, zeros_hbm, out_hbm,
             src_v, dst_v, ev_v, rows_v, acc, sem):
        c = lax.axis_index("c")
        s = lax.axis_index("s")
        wid = c * NUM_SUBCORES + s

        r0 = s * ROWS_PER_TILE
        pltpu.sync_copy(zeros_hbm.at[pl.ds(r0, ROWS_PER_TILE)],
                        acc.at[pl.ds(r0, ROWS_PER_TILE)])
        plsc.subcore_barrier()

        tile_base = wid * per_tile

        def chunk_body(i, carry):
            base = tile_base + i * K
            pltpu.sync_copy(src_hbm.at[pl.ds(base, K)], src_v)
            pltpu.sync_copy(dst_hbm.at[pl.ds(base, K)], dst_v)
            pltpu.sync_copy(ev_hbm.at[pl.ds(base, K)], ev_v)
            pltpu.async_copy(xw_hbm.at[src_v], rows_v, sem).wait()
            for g in range(K // 16):
                ev_g = ev_v[pl.ds(g * 16, 16)]
                for t in range(16):
                    e = g * 16 + t
                    scal = ev_g[t]
                    for j in range(D // 16):
                        sl = rows_v[e, pl.ds(j * 16, 16)]
                        rows_v[e, pl.ds(j * 16, 16)] = sl * scal
            pltpu.sync_copy(rows_v, acc.at[dst_v], add=True)
            return carry

        lax.fori_loop(0, n_chunks, chunk_body, 0)
        plsc.subcore_barrier()
        pltpu.sync_copy(acc.at[pl.ds(r0, ROWS_PER_TILE)],
                        out_hbm.at[c, pl.ds(r0, ROWS_PER_TILE)])

    return spmm


def kernel(x, edge_index, edge_values, weight):
    n, d_in = x.shape
    d_out = weight.shape[1]

    xw = pl.pallas_call(
        _matmul_body,
        grid=(5,),
        in_specs=[
            pl.BlockSpec((n // 5, d_in), lambda i: (i, 0)),
            pl.BlockSpec((d_in, d_out), lambda i: (0, 0)),
        ],
        out_specs=pl.BlockSpec((n // 5, d_out), lambda i: (i, 0)),
        out_shape=jax.ShapeDtypeStruct((n, d_out), jnp.float32),
    )(x, weight)

    src = edge_index[0].astype(jnp.int32)
    dst = edge_index[1].astype(jnp.int32)
    ev = edge_values.astype(jnp.float32)

    e = src.shape[0]
    chunk_stride = NW * K
    n_chunks = -(-e // chunk_stride)
    e_pad = n_chunks * chunk_stride
    if e_pad != e:
        pad = e_pad - e
        src = jnp.concatenate([src, jnp.zeros((pad,), jnp.int32)])
        dst = jnp.concatenate([dst, jnp.zeros((pad,), jnp.int32)])
        ev = jnp.concatenate([ev, jnp.zeros((pad,), jnp.float32)])
    per_tile = e_pad // NW

    zeros = jnp.zeros((N_PAD, d_out), jnp.float32)
    partials = _make_spmm(n_chunks, per_tile)(src, dst, ev, xw, zeros)

    out = pl.pallas_call(
        _combine_body,
        grid=(5,),
        in_specs=[
            pl.BlockSpec((NUM_CORES, n // 5, d_out), lambda i: (0, i, 0)),
        ],
        out_specs=pl.BlockSpec((n // 5, d_out), lambda i: (i, 0)),
        out_shape=jax.ShapeDtypeStruct((n, d_out), jnp.float32),
    )(partials)
    return out


# speedup vs baseline: 6.6283x; 1.6732x over previous
import functools

import jax
import jax.numpy as jnp
from jax import lax
from jax.experimental import pallas as pl
from jax.experimental.pallas import tpu as pltpu
from jax.experimental.pallas import tpu_sc as plsc

N_NODES = 10000
N_PAD = 10240
D = 128
K = 112
NB = 3
NUM_CORES = 2
NUM_SUBCORES = 16
NW = NUM_CORES * NUM_SUBCORES
ROWS_PER_TILE = N_PAD // NUM_SUBCORES


def _matmul_body(x_ref, w_ref, o_ref):
    o_ref[...] = jnp.dot(x_ref[...], w_ref[...],
                         preferred_element_type=jnp.float32)


def _combine_body(p_ref, o_ref):
    o_ref[...] = p_ref[0] + p_ref[1]


@functools.lru_cache(maxsize=None)
def _make_spmm(n_chunks):
    mesh = plsc.VectorSubcoreMesh(core_axis_name="c", subcore_axis_name="s")

    @functools.partial(
        pl.kernel,
        out_type=jax.ShapeDtypeStruct((NUM_CORES, N_PAD, D), jnp.float32),
        mesh=mesh,
        scratch_types=[
            pltpu.VMEM((NB, 2, K), jnp.int32),
            pltpu.VMEM((NB, K), jnp.float32),
            pltpu.VMEM((NB, K, D), jnp.float32),
            pltpu.VMEM_SHARED((N_PAD, D), jnp.float32),
            [pltpu.SemaphoreType.DMA] * NB,
            [pltpu.SemaphoreType.DMA] * NB,
            [pltpu.SemaphoreType.DMA] * NB,
        ],
    )
    def spmm(p_hbm, ev_hbm, xw_hbm, zeros_hbm, out_hbm,
             idx_v, ev_v, rows_v, acc, sem_i, sem_g, sem_s):
        c = lax.axis_index("c")
        s = lax.axis_index("s")
        wid = c * NUM_SUBCORES + s

        r0 = s * ROWS_PER_TILE
        pltpu.sync_copy(zeros_hbm.at[pl.ds(r0, ROWS_PER_TILE)],
                        acc.at[pl.ds(r0, ROWS_PER_TILE)])
        plsc.subcore_barrier()

        cbase = wid * n_chunks

        def issue_idx(i, r):
            pltpu.async_copy(p_hbm.at[cbase + i], idx_v.at[r], sem_i[r])
            pltpu.async_copy(ev_hbm.at[pl.ds((cbase + i) * K, K)],
                             ev_v.at[r], sem_i[r])

        def wait_idx(r):
            pltpu.make_async_copy(p_hbm.at[0], idx_v.at[r], sem_i[r]).wait()
            pltpu.make_async_copy(ev_hbm.at[pl.ds(0, K)], ev_v.at[r],
                                  sem_i[r]).wait()

        def issue_gather(r):
            pltpu.async_copy(xw_hbm.at[idx_v.at[r, 0]], rows_v.at[r],
                             sem_g[r])

        def wait_gather(r):
            pltpu.make_async_copy(xw_hbm.at[idx_v.at[r, 0]], rows_v.at[r],
                                  sem_g[r]).wait()

        def issue_scatter(r):
            pltpu.async_copy(rows_v.at[r], acc.at[idx_v.at[r, 1]],
                             sem_s[r], add=True)

        def wait_scatter(r):
            pltpu.make_async_copy(rows_v.at[r], acc.at[idx_v.at[r, 1]],
                                  sem_s[r]).wait()

        def scale(r):
            def g_body(g, carry):
                evf = ev_v[r, pl.ds(g * 16, 16)]
                for t in range(16):
                    scal = evf[t]
                    e = g * 16 + t
                    for j in range(D // 16):
                        sl = rows_v[r, e, pl.ds(j * 16, 16)]
                        rows_v[r, e, pl.ds(j * 16, 16)] = sl * scal
                return carry

            lax.fori_loop(0, K // 16, g_body, 0)

        issue_idx(0, 0)
        issue_idx(1, 1)
        wait_idx(0)
        issue_gather(0)

        def outer(k, carry):
            i0 = k * NB
            for r in range(NB):
                i = i0 + r
                r1 = (r + 1) % NB
                r2 = (r + 2) % NB
                wait_idx(r1)
                issue_gather(r1)
                wait_gather(r)
                scale(r)
                @pl.when(i > 0)
                def _():
                    wait_scatter(r2)
                issue_scatter(r)
                issue_idx(i + 2, r2)
            return carry

        lax.fori_loop(0, n_chunks // NB, outer, 0)

        wait_idx((n_chunks + 1) % NB)
        wait_gather(n_chunks % NB)
        wait_scatter((n_chunks - 1) % NB)

        plsc.subcore_barrier()
        pltpu.sync_copy(acc.at[pl.ds(r0, ROWS_PER_TILE)],
                        out_hbm.at[c, pl.ds(r0, ROWS_PER_TILE)])

    return spmm


def kernel(x, edge_index, edge_values, weight):
    n, d_in = x.shape
    d_out = weight.shape[1]

    xw = pl.pallas_call(
        _matmul_body,
        grid=(5,),
        in_specs=[
            pl.BlockSpec((n // 5, d_in), lambda i: (i, 0)),
            pl.BlockSpec((d_in, d_out), lambda i: (0, 0)),
        ],
        out_specs=pl.BlockSpec((n // 5, d_out), lambda i: (i, 0)),
        out_shape=jax.ShapeDtypeStruct((n, d_out), jnp.float32),
    )(x, weight)

    src = edge_index[0].astype(jnp.int32)
    dst = edge_index[1].astype(jnp.int32)
    ev = edge_values.astype(jnp.float32)

    e = src.shape[0]
    chunk_stride = NW * K * NB
    n_chunks = NB * (-(-e // chunk_stride))
    e_pad = n_chunks * NW * K
    if e_pad != e:
        pad = e_pad - e
        src = jnp.concatenate([src, jnp.zeros((pad,), jnp.int32)])
        dst = jnp.concatenate([dst, jnp.zeros((pad,), jnp.int32)])
        ev = jnp.concatenate([ev, jnp.zeros((pad,), jnp.float32)])

    packed = jnp.stack([src.reshape(-1, K), dst.reshape(-1, K)], axis=1)
    packed = jnp.concatenate(
        [packed, jnp.zeros((2, 2, K), jnp.int32)], axis=0)
    ev = jnp.concatenate([ev, jnp.zeros((2 * K,), jnp.float32)])

    zeros = jnp.zeros((N_PAD, d_out), jnp.float32)
    partials = _make_spmm(n_chunks)(packed, ev, xw, zeros)

    out = pl.pallas_call(
        _combine_body,
        grid=(5,),
        in_specs=[
            pl.BlockSpec((NUM_CORES, n // 5, d_out), lambda i: (0, i, 0)),
        ],
        out_specs=pl.BlockSpec((n // 5, d_out), lambda i: (i, 0)),
        out_shape=jax.ShapeDtypeStruct((n, d_out), jnp.float32),
    )(partials)
    return out
